# baseline (speedup 1.0000x reference)
import functools

import jax
import jax.numpy as jnp
from jax.experimental import pallas as pl


def _ffn_body(x_ref, tw_ref, ti_ref, w1_ref, w2_ref, out_ref):
    e = pl.program_id(0)
    fi = pl.program_id(1)

    x = x_ref[...].astype(jnp.bfloat16)
    wg = w1_ref[0, 0].astype(jnp.bfloat16)
    wu = w1_ref[0, 1].astype(jnp.bfloat16)
    g = jax.lax.dot_general(x, wg, (((1,), (1,)), ((), ())),
                            preferred_element_type=jnp.float32)
    u = jax.lax.dot_general(x, wu, (((1,), (1,)), ((), ())),
                            preferred_element_type=jnp.float32)
    act = (g * jax.nn.sigmoid(g)) * u
    w2c = w2_ref[0].astype(jnp.bfloat16)
    y = jax.lax.dot_general(act.astype(jnp.bfloat16), w2c,
                            (((1,), (1,)), ((), ())),
                            preferred_element_type=jnp.float32)

    ids = ti_ref[...]
    tw = tw_ref[...]
    wvec = jnp.sum(jnp.where(ids == e, tw, 0.0), axis=1)

    @pl.when((e == 0) & (fi == 0))
    def _():
        out_ref[...] = jnp.zeros_like(out_ref)

    out_ref[...] += wvec[:, None] * y


@jax.jit
def kernel(hidden_states, topk_weights, topk_ids, w1, w2):
    T, D = hidden_states.shape
    E = w1.shape[0]
    F = w1.shape[1] // 2
    FC = 512
    NF = F // FC

    w1r = w1.reshape(E, 2, F, D)

    grid = (E, NF)
    out = pl.pallas_call(
        _ffn_body,
        grid=grid,
        in_specs=[
            pl.BlockSpec((T, D), lambda e, f: (0, 0)),
            pl.BlockSpec(topk_weights.shape, lambda e, f: (0, 0)),
            pl.BlockSpec(topk_ids.shape, lambda e, f: (0, 0)),
            pl.BlockSpec((1, 2, FC, D), lambda e, f: (e, 0, f, 0)),
            pl.BlockSpec((1, D, FC), lambda e, f: (e, 0, f)),
        ],
        out_specs=pl.BlockSpec((T, D), lambda e, f: (0, 0)),
        out_shape=jax.ShapeDtypeStruct((T, D), jnp.float32),
    )(hidden_states, topk_weights, topk_ids, w1r, w2)
    return out


# speedup vs baseline: 1.0994x; 1.0994x over previous
import functools

import jax
import jax.numpy as jnp
from jax.experimental import pallas as pl


def _ffn_body(x_ref, tw_ref, ti_ref, w1_ref, w2_ref, out_ref):
    e = pl.program_id(0)
    fi = pl.program_id(1)

    x = x_ref[...]
    wg = w1_ref[0, 0]
    wu = w1_ref[0, 1]
    g = jax.lax.dot_general(x, wg, (((1,), (1,)), ((), ())),
                            preferred_element_type=jnp.float32)
    u = jax.lax.dot_general(x, wu, (((1,), (1,)), ((), ())),
                            preferred_element_type=jnp.float32)
    act = (g * jax.nn.sigmoid(g)) * u
    w2c = w2_ref[0]
    y = jax.lax.dot_general(act, w2c, (((1,), (1,)), ((), ())),
                            preferred_element_type=jnp.float32)

    ids = ti_ref[...]
    tw = tw_ref[...]
    wvec = jnp.sum(jnp.where(ids == e, tw, 0.0), axis=1)

    @pl.when((e == 0) & (fi == 0))
    def _():
        out_ref[...] = jnp.zeros_like(out_ref)

    out_ref[...] += wvec[:, None] * y


@jax.jit
def kernel(hidden_states, topk_weights, topk_ids, w1, w2):
    T, D = hidden_states.shape
    E = w1.shape[0]
    F = w1.shape[1] // 2
    FC = 1024
    NF = F // FC

    w1r = w1.reshape(E, 2, F, D)

    grid = (E, NF)
    out = pl.pallas_call(
        _ffn_body,
        grid=grid,
        in_specs=[
            pl.BlockSpec((T, D), lambda e, f: (0, 0)),
            pl.BlockSpec(topk_weights.shape, lambda e, f: (0, 0)),
            pl.BlockSpec(topk_ids.shape, lambda e, f: (0, 0)),
            pl.BlockSpec((1, 2, FC, D), lambda e, f: (e, 0, f, 0)),
            pl.BlockSpec((1, D, FC), lambda e, f: (e, 0, f)),
        ],
        out_specs=pl.BlockSpec((T, D), lambda e, f: (0, 0)),
        out_shape=jax.ShapeDtypeStruct((T, D), jnp.float32),
    )(hidden_states, topk_weights, topk_ids, w1r, w2)
    return out
